# Initial kernel scaffold; baseline (speedup 1.0000x reference)
import jax
import jax.numpy as jnp
from jax.experimental import pallas as pl


def kernel(x, X, Y):
    raise NotImplementedError("write your pallas kernel here")


# speedup vs baseline: 461.0971x; 461.0971x over previous
import functools

import jax
import jax.numpy as jnp
from jax import lax
from jax.experimental import pallas as pl
from jax.experimental.pallas import tpu as pltpu
from jax.experimental.pallas import tpu_sc as plsc

N_LOC = 100000
BATCH = 16384
SEQ = 200
OUT_PER_ROW = SEQ - 1
NC, NS, LANES = 2, 16, 16
NW = NC * NS
ROWS_PER_W = BATCH // NW
ROWS_PER_GRP = 8
GRPS_PER_W = ROWS_PER_W // ROWS_PER_GRP
IDX_PER_GRP = ROWS_PER_GRP * SEQ
OUT_PER_GRP = ROWS_PER_GRP * OUT_PER_ROW
CHUNKS = (OUT_PER_ROW + LANES - 1) // LANES


def _body(x_hbm, tab_hbm, out_hbm, tab_v, idx_v, out_v):
    w = lax.axis_index("c") * NS + lax.axis_index("s")

    pltpu.sync_copy(tab_hbm, tab_v)

    ii = lax.iota(jnp.int32, LANES)

    def per_group(g, carry):
        gi = w * GRPS_PER_W + g
        pltpu.sync_copy(x_hbm.at[pl.ds(gi * IDX_PER_GRP, IDX_PER_GRP)], idx_v)

        def per_row(r, carry2):
            base = r * SEQ
            obase = r * OUT_PER_ROW
            for c in range(CHUNKS):
                t = ii + (c * LANES)
                if (c + 1) * LANES <= OUT_PER_ROW:
                    tc = t
                    mask = None
                else:
                    tc = jnp.minimum(t, OUT_PER_ROW - 1)
                    mask = t < OUT_PER_ROW
                f1 = tc + base
                i1 = plsc.load_gather(idx_v, [f1])
                i2 = plsc.load_gather(idx_v, [f1 + 1])
                v1 = plsc.load_gather(tab_v, [i1])
                v2 = plsc.load_gather(tab_v, [i2])
                d = v1 - v2
                plsc.store_scatter(out_v, [t + obase], d * d, mask=mask)
            return carry2

        lax.fori_loop(0, ROWS_PER_GRP, per_row, 0)
        pltpu.sync_copy(out_v, out_hbm.at[pl.ds(gi * OUT_PER_GRP, OUT_PER_GRP)])
        return carry

    lax.fori_loop(0, GRPS_PER_W, per_group, 0)


@jax.jit
def _run(x_flat, X):
    mesh = plsc.VectorSubcoreMesh(core_axis_name="c", subcore_axis_name="s")
    f = functools.partial(
        pl.kernel,
        out_type=jax.ShapeDtypeStruct((BATCH * OUT_PER_ROW,), jnp.float32),
        mesh=mesh,
        scratch_types=[
            pltpu.VMEM((N_LOC,), jnp.float32),
            pltpu.VMEM((IDX_PER_GRP,), jnp.int32),
            pltpu.VMEM((OUT_PER_GRP,), jnp.float32),
        ],
        compiler_params=pltpu.CompilerParams(needs_layout_passes=False),
    )(_body)
    return f(x_flat, X)


def kernel(x, X, Y):
    del Y
    x_flat = x.reshape(-1).astype(jnp.int32)
    return _run(x_flat, X)
